# baseline (speedup 1.0000x reference)
import functools

import jax
import jax.numpy as jnp
from jax import lax
from jax.experimental import pallas as pl
from jax.experimental.pallas import tpu as pltpu
from jax.experimental.pallas import tpu_sc as plsc

H = 128
IN = 4
NC, NS = 2, 16
NW = NC * NS
CHUNK = 128
DW = 16

@functools.cache
def _sc_mesh():
    return plsc.VectorSubcoreMesh(
        core_axis_name="c", subcore_axis_name="s",
        num_cores=NC, num_subcores=NS)


def _worker(Kw):
    c = lax.axis_index("c")
    s = lax.axis_index("s")
    return c, s, (c * NS + s) * Kw


def _make_hist_kernel(n_chunks, n_pad):
    Kw = n_chunks // NW

    @functools.partial(
        pl.kernel,
        out_type=jax.ShapeDtypeStruct((NC, n_pad, H), jnp.float32),
        mesh=_sc_mesh(),
        scratch_types=[
            pltpu.VMEM((Kw, CHUNK), jnp.int32),
            pltpu.VMEM((CHUNK, H), jnp.float32),
            pltpu.VMEM_SHARED((n_pad, H), jnp.float32),
        ],
    )
    def hist(col_hbm, ones_hbm, zeros_hbm, out_hbm, idx_v, ones_v, acc_sh):
        c, s, base = _worker(Kw)
        pltpu.sync_copy(col_hbm.at[pl.ds(base, Kw)], idx_v)
        pltpu.sync_copy(ones_hbm, ones_v)

        @pl.when(s == 0)
        def _():
            pltpu.sync_copy(zeros_hbm, acc_sh)

        plsc.subcore_barrier()

        def body(j, carry):
            pltpu.sync_copy(ones_v, acc_sh.at[idx_v.at[j]], add=True)
            return carry

        lax.fori_loop(0, Kw, body, 0)
        plsc.subcore_barrier()

        @pl.when(s == 0)
        def _():
            pltpu.sync_copy(acc_sh, out_hbm.at[c])

    return hist


def _make_gather_kernel(n_chunks, e_pad, dtype, frac0=0.5):
    NB = 2
    k0 = int(n_chunks * frac0 / (NS * NB)) * NB
    k1 = n_chunks // NS - k0
    kmax = max(k0, k1)

    @functools.partial(
        pl.kernel,
        out_type=(
            jax.ShapeDtypeStruct((e_pad, H), dtype),
            jax.ShapeDtypeStruct((e_pad, H), dtype),
        ),
        mesh=_sc_mesh(),
        scratch_types=(
            [pltpu.VMEM((kmax, CHUNK), jnp.int32)] * 2
            + [pltpu.VMEM((CHUNK, H), dtype)] * (2 * NB)
            + [pltpu.SemaphoreType.DMA] * (2 * NB)
        ),
    )
    def gather(hid_hbm, row_hbm, col_hbm, outr_hbm, outc_hbm,
               rowi_v, coli_v, *rest):
        bufr = rest[0:NB]
        bufc = rest[NB:2 * NB]
        semr = rest[2 * NB:3 * NB]
        semc = rest[3 * NB:4 * NB]
        c = lax.axis_index("c")
        s = lax.axis_index("s")

        def run(base, Kw):
            pltpu.sync_copy(row_hbm.at[pl.ds(base, Kw)],
                            rowi_v.at[pl.ds(0, Kw)])
            pltpu.sync_copy(col_hbm.at[pl.ds(base, Kw)],
                            coli_v.at[pl.ds(0, Kw)])

            for b in range(NB):
                pltpu.async_copy(hid_hbm.at[rowi_v.at[b]], bufr[b], semr[b])
                pltpu.async_copy(hid_hbm.at[coli_v.at[b]], bufc[b], semc[b])

            def body(i, carry):
                j0 = i * NB
                for b in range(NB):
                    j = j0 + b
                    off = (base + j) * CHUNK
                    pltpu.make_async_copy(hid_hbm.at[rowi_v.at[j]],
                                          bufr[b], semr[b]).wait()
                    pltpu.sync_copy(bufr[b], outr_hbm.at[pl.ds(off, CHUNK)])
                    pltpu.make_async_copy(hid_hbm.at[coli_v.at[j]],
                                          bufc[b], semc[b]).wait()
                    pltpu.sync_copy(bufc[b], outc_hbm.at[pl.ds(off, CHUNK)])

                    @pl.when(j + NB < Kw)
                    def _():
                        pltpu.async_copy(hid_hbm.at[rowi_v.at[j + NB]],
                                         bufr[b], semr[b])
                        pltpu.async_copy(hid_hbm.at[coli_v.at[j + NB]],
                                         bufc[b], semc[b])
                return carry

            lax.fori_loop(0, Kw // NB, body, 0)

        if k0 > 0:
            @pl.when(c == 0)
            def _():
                run(s * k0, k0)
        if k1 > 0:
            @pl.when(c == 1)
            def _():
                run(NS * k0 + s * k1, k1)

    return gather


def _make_scatter_kernel(n_chunks, n_pad):
    Kw = n_chunks // NW

    @functools.partial(
        pl.kernel,
        out_type=jax.ShapeDtypeStruct((NC, n_pad, H), jnp.float32),
        mesh=_sc_mesh(),
        scratch_types=[
            pltpu.VMEM((Kw, CHUNK), jnp.int32),
            pltpu.VMEM((CHUNK, H), jnp.float32),
            pltpu.VMEM((CHUNK, H), jnp.float32),
            pltpu.SemaphoreType.DMA,
            pltpu.SemaphoreType.DMA,
            pltpu.VMEM_SHARED((n_pad, H), jnp.float32),
        ],
    )
    def scatter(msg_hbm, col_hbm, zeros_hbm, out_hbm, idx_v, buf0, buf1,
                sem0, sem1, acc_sh):
        bufs = (buf0, buf1)
        sems = (sem0, sem1)
        c, s, base = _worker(Kw)
        pltpu.sync_copy(col_hbm.at[pl.ds(base, Kw)], idx_v)

        @pl.when(s == 0)
        def _():
            pltpu.sync_copy(zeros_hbm, acc_sh)

        plsc.subcore_barrier()
        pltpu.async_copy(msg_hbm.at[pl.ds(base * CHUNK, CHUNK)], bufs[0], sems[0])

        def body(i, carry):
            j0 = i * 2
            for b in range(2):
                j = j0 + b
                pltpu.make_async_copy(
                    msg_hbm.at[pl.ds((base + j) * CHUNK, CHUNK)],
                    bufs[b], sems[b]).wait()

                @pl.when(j + 1 < Kw)
                def _():
                    pltpu.async_copy(
                        msg_hbm.at[pl.ds((base + j + 1) * CHUNK, CHUNK)],
                        bufs[1 - b], sems[1 - b])

                pltpu.sync_copy(bufs[b], acc_sh.at[idx_v.at[j]], add=True)
            return carry

        lax.fori_loop(0, Kw // 2, body, 0)
        plsc.subcore_barrier()

        @pl.when(s == 0)
        def _():
            pltpu.sync_copy(acc_sh, out_hbm.at[c])

    return scatter


def _sigmoid(x):
    return jax.nn.sigmoid(x)


def _step0_body(x_ref, dega_ref, degb_ref,
                b1_ref, w2_ref, b2_ref,
                whr_ref, whi_ref, whh_ref,
                wir_ref, bir_ref, wii_ref, bii_ref, win_ref, bin_ref,
                o1_ref, ob1_ref, o2_ref, ob2_ref, o3_ref, ob3_ref,
                h1_ref, pred_ref):
    cvec = None
    for i in range(3):
        m = jnp.tanh(b1_ref[i:i + 1, :])
        m = jnp.tanh(jnp.dot(m, w2_ref[i]) + b2_ref[i:i + 1, :])
        cvec = m if cvec is None else cvec + m
    cr = jnp.dot(cvec, whr_ref[...])
    ci = jnp.dot(cvec, whi_ref[...])
    ch = jnp.dot(cvec, whh_ref[...])
    d = dega_ref[:, 0:1] + degb_ref[:, 0:1]
    x = x_ref[...]
    r = _sigmoid(jnp.dot(x, wir_ref[...]) + bir_ref[...] + d * cr)
    ii = _sigmoid(jnp.dot(x, wii_ref[...]) + bii_ref[...] + d * ci)
    nn = jnp.tanh(jnp.dot(x, win_ref[...]) + bin_ref[...] + r * (d * ch))
    h1 = (1.0 - ii) * nn
    h1_ref[...] = h1
    p = jax.nn.relu(jnp.dot(h1, o1_ref[...]) + ob1_ref[...])
    p = jax.nn.relu(jnp.dot(p, o2_ref[...]) + ob2_ref[...])
    p = jax.nn.relu(jnp.dot(p, o3_ref[...]) + ob3_ref[...])
    pred_ref[...] = x + p


def _edge_mlp_body(hrow_ref, hcol_ref, w1r_ref, w1c_ref, b1_ref,
                   w2_ref, b2_ref, out_ref):
    f32 = jnp.float32
    hr = hrow_ref[...].astype(jnp.bfloat16)
    hc = hcol_ref[...].astype(jnp.bfloat16)
    u = (jnp.dot(hr, w1r_ref[...], preferred_element_type=f32)
         + jnp.dot(hc, w1c_ref[...], preferred_element_type=f32)
         + b1_ref[...])
    m = jnp.tanh(u).astype(jnp.bfloat16)
    acc = None
    for i in range(3):
        t = jnp.tanh(jnp.dot(m[:, i * H:(i + 1) * H], w2_ref[i],
                             preferred_element_type=f32) + b2_ref[i:i + 1, :])
        acc = t if acc is None else acc + t
    out_ref[...] = acc


def _step1_body(x_ref, h_ref, agga_ref, aggb_ref,
                whr_ref, whi_ref, whh_ref,
                wir_ref, bir_ref, wii_ref, bii_ref, win_ref, bin_ref,
                o1_ref, ob1_ref, o2_ref, ob2_ref, o3_ref, ob3_ref,
                pred_ref):
    agg = agga_ref[...] + aggb_ref[...]
    x = x_ref[...]
    h = h_ref[...]
    r = _sigmoid(jnp.dot(x, wir_ref[...]) + bir_ref[...] + jnp.dot(agg, whr_ref[...]))
    ii = _sigmoid(jnp.dot(x, wii_ref[...]) + bii_ref[...] + jnp.dot(agg, whi_ref[...]))
    nn = jnp.tanh(jnp.dot(x, win_ref[...]) + bin_ref[...]
                  + r * jnp.dot(agg, whh_ref[...]))
    h2 = (1.0 - ii) * nn + ii * h
    p = jax.nn.relu(jnp.dot(h2, o1_ref[...]) + ob1_ref[...])
    p = jax.nn.relu(jnp.dot(p, o2_ref[...]) + ob2_ref[...])
    p = jax.nn.relu(jnp.dot(p, o3_ref[...]) + ob3_ref[...])
    pred_ref[...] = x + p


def _full(shape):
    return pl.BlockSpec(shape, lambda i: (0,) * len(shape))


def _rows(bn, w):
    return pl.BlockSpec((bn, w), lambda i: (i, 0))


def kernel(inputs, edge_index, burn_in_steps, params):
    _, _, N, _ = inputs.shape
    E = edge_index.shape[1]
    p = params

    e_pad = ((E + NW * CHUNK - 1) // (NW * CHUNK)) * (NW * CHUNK)
    n_chunks = e_pad // CHUNK
    n_pad = ((N + DW) + 63) // 64 * 64
    pad = e_pad - E

    row = edge_index[0]
    col = edge_index[1]
    row_p = jnp.concatenate([row, jnp.zeros((pad,), jnp.int32)]).reshape(n_chunks, CHUNK)
    col_p = jnp.concatenate([col, jnp.full((pad,), N, jnp.int32)]).reshape(n_chunks, CHUNK)

    ones_h = jnp.ones((CHUNK, H), jnp.float32)
    zeros_nh = jnp.zeros((n_pad, H), jnp.float32)

    deg2 = _make_hist_kernel(n_chunks, n_pad)(col_p, ones_h, zeros_nh)
    dega = deg2[0, :N, :DW]
    degb = deg2[1, :N, :DW]

    x0 = inputs[0, 0]

    BN = 2000
    grid = (N // BN,)
    wb = [
        p["msg_fc1_b"], p["msg_fc2_W"], p["msg_fc2_b"],
        p["hidden_r_W"], p["hidden_i_W"], p["hidden_h_W"],
        p["input_r_W"], p["input_r_b"].reshape(1, H),
        p["input_i_W"], p["input_i_b"].reshape(1, H),
        p["input_n_W"], p["input_n_b"].reshape(1, H),
        p["out_fc1_W"], p["out_fc1_b"].reshape(1, H),
        p["out_fc2_W"], p["out_fc2_b"].reshape(1, H),
        p["out_fc3_W"], p["out_fc3_b"].reshape(1, IN),
    ]
    wb_specs = [_full(w.shape) for w in wb]
    h1, pred0 = pl.pallas_call(
        _step0_body,
        grid=grid,
        in_specs=[_rows(BN, IN), _rows(BN, DW), _rows(BN, DW)] + wb_specs,
        out_specs=[_rows(BN, H), _rows(BN, IN)],
        out_shape=[
            jax.ShapeDtypeStruct((N, H), jnp.float32),
            jax.ShapeDtypeStruct((N, IN), jnp.float32),
        ],
    )(x0, dega, degb, *wb)

    hrow, hcol = _make_gather_kernel(n_chunks, e_pad, jnp.float32, 1.0)(
        h1, row_p, col_p)

    bf16 = jnp.bfloat16
    w1 = p["msg_fc1_W"]
    w1r = jnp.concatenate([w1[i, :H, :] for i in range(3)], axis=1).astype(bf16)
    w1c = jnp.concatenate([w1[i, H:, :] for i in range(3)], axis=1).astype(bf16)
    b1f = p["msg_fc1_b"].reshape(1, 3 * H)
    w2b = p["msg_fc2_W"].astype(bf16)
    BE = 1280
    msgs = pl.pallas_call(
        _edge_mlp_body,
        grid=(e_pad // BE,),
        in_specs=[_rows(BE, H), _rows(BE, H), _full((H, 3 * H)),
                  _full((H, 3 * H)), _full((1, 3 * H)),
                  _full((3, H, H)), _full((3, H))],
        out_specs=[_rows(BE, H)],
        out_shape=[jax.ShapeDtypeStruct((e_pad, H), jnp.float32)],
    )(hrow, hcol, w1r, w1c, b1f, w2b, p["msg_fc2_b"])[0]

    agg2 = _make_scatter_kernel(n_chunks, n_pad)(msgs, col_p, zeros_nh)
    agga = agg2[0, :N, :]
    aggb = agg2[1, :N, :]

    x1 = jnp.where(jnp.asarray(burn_in_steps) > 1, inputs[0, 1], pred0)
    wb1 = wb[3:]
    wb1_specs = [_full(w.shape) for w in wb1]
    pred1 = pl.pallas_call(
        _step1_body,
        grid=grid,
        in_specs=[_rows(BN, IN), _rows(BN, H), _rows(BN, H), _rows(BN, H)]
        + wb1_specs,
        out_specs=[_rows(BN, IN)],
        out_shape=[jax.ShapeDtypeStruct((N, IN), jnp.float32)],
    )(x1, h1, agga, aggb, *wb1)[0]

    return jnp.stack([pred0, pred1], axis=0)[None]


# speedup vs baseline: 1.1363x; 1.1363x over previous
import functools

import jax
import jax.numpy as jnp
from jax import lax
from jax.experimental import pallas as pl
from jax.experimental.pallas import tpu as pltpu
from jax.experimental.pallas import tpu_sc as plsc

H = 128
IN = 4
NC, NS = 2, 16
NW = NC * NS
CHUNK = 128
DW = 16

@functools.cache
def _sc_mesh():
    return plsc.VectorSubcoreMesh(
        core_axis_name="c", subcore_axis_name="s",
        num_cores=NC, num_subcores=NS)


def _worker(Kw):
    c = lax.axis_index("c")
    s = lax.axis_index("s")
    return c, s, (c * NS + s) * Kw


def _make_hist_kernel(n_chunks, n_pad):
    Kw = n_chunks // NW

    @functools.partial(
        pl.kernel,
        out_type=jax.ShapeDtypeStruct((NC, n_pad, H), jnp.float32),
        mesh=_sc_mesh(),
        scratch_types=[
            pltpu.VMEM((Kw, CHUNK), jnp.int32),
            pltpu.VMEM((CHUNK, H), jnp.float32),
            pltpu.VMEM_SHARED((n_pad, H), jnp.float32),
        ],
    )
    def hist(col_hbm, ones_hbm, zeros_hbm, out_hbm, idx_v, ones_v, acc_sh):
        c, s, base = _worker(Kw)
        pltpu.sync_copy(col_hbm.at[pl.ds(base, Kw)], idx_v)
        pltpu.sync_copy(ones_hbm, ones_v)

        @pl.when(s == 0)
        def _():
            pltpu.sync_copy(zeros_hbm, acc_sh)

        plsc.subcore_barrier()

        def body(j, carry):
            pltpu.sync_copy(ones_v, acc_sh.at[idx_v.at[j]], add=True)
            return carry

        lax.fori_loop(0, Kw, body, 0)
        plsc.subcore_barrier()

        @pl.when(s == 0)
        def _():
            pltpu.sync_copy(acc_sh, out_hbm.at[c])

    return hist


def _make_gather_kernel(n_chunks, e_pad, width, frac0=0.5, chunk0=0):
    NB = 2
    dtype = jnp.float32
    k0 = int(round(n_chunks * frac0 / (NS * 8))) * 8
    k1 = n_chunks // NS - k0
    kmax = max(k0, k1)

    @functools.partial(
        pl.kernel,
        out_type=(
            jax.ShapeDtypeStruct((e_pad, width), dtype),
            jax.ShapeDtypeStruct((e_pad, width), dtype),
        ),
        mesh=_sc_mesh(),
        scratch_types=(
            [pltpu.VMEM((kmax, CHUNK), jnp.int32)] * 2
            + [pltpu.VMEM((CHUNK, width), dtype)] * (2 * NB)
            + [pltpu.SemaphoreType.DMA] * (2 * NB)
        ),
    )
    def gather(hid_hbm, row_hbm, col_hbm, outr_hbm, outc_hbm,
               rowi_v, coli_v, *rest):
        bufr = rest[0:NB]
        bufc = rest[NB:2 * NB]
        semr = rest[2 * NB:3 * NB]
        semc = rest[3 * NB:4 * NB]
        c = lax.axis_index("c")
        s = lax.axis_index("s")

        def run(base, Kw):
            pltpu.sync_copy(row_hbm.at[pl.ds(chunk0 + base, Kw)],
                            rowi_v.at[pl.ds(0, Kw)])
            pltpu.sync_copy(col_hbm.at[pl.ds(chunk0 + base, Kw)],
                            coli_v.at[pl.ds(0, Kw)])

            for b in range(NB):
                pltpu.async_copy(hid_hbm.at[rowi_v.at[b]], bufr[b], semr[b])
                pltpu.async_copy(hid_hbm.at[coli_v.at[b]], bufc[b], semc[b])

            def body(i, carry):
                j0 = i * NB
                for b in range(NB):
                    j = j0 + b
                    off = (base + j) * CHUNK
                    pltpu.make_async_copy(hid_hbm.at[rowi_v.at[j]],
                                          bufr[b], semr[b]).wait()
                    pltpu.sync_copy(bufr[b], outr_hbm.at[pl.ds(off, CHUNK)])
                    pltpu.make_async_copy(hid_hbm.at[coli_v.at[j]],
                                          bufc[b], semc[b]).wait()
                    pltpu.sync_copy(bufc[b], outc_hbm.at[pl.ds(off, CHUNK)])

                    @pl.when(j + NB < Kw)
                    def _():
                        pltpu.async_copy(hid_hbm.at[rowi_v.at[j + NB]],
                                         bufr[b], semr[b])
                        pltpu.async_copy(hid_hbm.at[coli_v.at[j + NB]],
                                         bufc[b], semc[b])
                return carry

            lax.fori_loop(0, Kw // NB, body, 0)

        if k0 > 0:
            @pl.when(c == 0)
            def _():
                run(s * k0, k0)
        if k1 > 0:
            @pl.when(c == 1)
            def _():
                run(NS * k0 + s * k1, k1)

    return gather


def _make_scatter_kernel(n_chunks, n_pad, chunk0=0):
    k0 = int(round(n_chunks / (2 * NS * 8))) * 8
    k1 = n_chunks // NS - k0
    kmax = max(k0, k1)

    @functools.partial(
        pl.kernel,
        out_type=jax.ShapeDtypeStruct((NC, n_pad, H), jnp.float32),
        mesh=_sc_mesh(),
        scratch_types=[
            pltpu.VMEM((kmax, CHUNK), jnp.int32),
            pltpu.VMEM((CHUNK, H), jnp.float32),
            pltpu.VMEM((CHUNK, H), jnp.float32),
            pltpu.SemaphoreType.DMA,
            pltpu.SemaphoreType.DMA,
            pltpu.VMEM_SHARED((n_pad, H), jnp.float32),
        ],
    )
    def scatter(msg_hbm, col_hbm, zeros_hbm, out_hbm, idx_v, buf0, buf1,
                sem0, sem1, acc_sh):
        bufs = (buf0, buf1)
        sems = (sem0, sem1)
        c = lax.axis_index("c")
        s = lax.axis_index("s")

        @pl.when(s == 0)
        def _():
            pltpu.sync_copy(zeros_hbm, acc_sh)

        plsc.subcore_barrier()

        def run(base, kw):
            pltpu.sync_copy(col_hbm.at[pl.ds(chunk0 + base, kw)],
                            idx_v.at[pl.ds(0, kw)])
            pltpu.async_copy(msg_hbm.at[pl.ds(base * CHUNK, CHUNK)],
                             bufs[0], sems[0])

            def body(i, carry):
                j0 = i * 2
                for b in range(2):
                    j = j0 + b
                    pltpu.make_async_copy(
                        msg_hbm.at[pl.ds((base + j) * CHUNK, CHUNK)],
                        bufs[b], sems[b]).wait()

                    @pl.when(j + 1 < kw)
                    def _():
                        pltpu.async_copy(
                            msg_hbm.at[pl.ds((base + j + 1) * CHUNK, CHUNK)],
                            bufs[1 - b], sems[1 - b])

                    pltpu.sync_copy(bufs[b], acc_sh.at[idx_v.at[j]], add=True)
                return carry

            lax.fori_loop(0, kw // 2, body, 0)

        if k0 > 0:
            @pl.when(c == 0)
            def _():
                run(s * k0, k0)
        if k1 > 0:
            @pl.when(c == 1)
            def _():
                run(NS * k0 + s * k1, k1)

        plsc.subcore_barrier()

        @pl.when(s == 0)
        def _():
            pltpu.sync_copy(acc_sh, out_hbm.at[c])

    return scatter


def _sigmoid(x):
    return jax.nn.sigmoid(x)


def _step0_body(x_ref, dega_ref, degb_ref,
                b1_ref, w2_ref, b2_ref,
                whr_ref, whi_ref, whh_ref,
                wir_ref, bir_ref, wii_ref, bii_ref, win_ref, bin_ref,
                o1_ref, ob1_ref, o2_ref, ob2_ref, o3_ref, ob3_ref,
                h1_ref, pred_ref):
    cvec = None
    for i in range(3):
        m = jnp.tanh(b1_ref[i:i + 1, :])
        m = jnp.tanh(jnp.dot(m, w2_ref[i]) + b2_ref[i:i + 1, :])
        cvec = m if cvec is None else cvec + m
    cr = jnp.dot(cvec, whr_ref[...])
    ci = jnp.dot(cvec, whi_ref[...])
    ch = jnp.dot(cvec, whh_ref[...])
    d = dega_ref[:, 0:1] + degb_ref[:, 0:1]
    x = x_ref[...]
    r = _sigmoid(jnp.dot(x, wir_ref[...]) + bir_ref[...] + d * cr)
    ii = _sigmoid(jnp.dot(x, wii_ref[...]) + bii_ref[...] + d * ci)
    nn = jnp.tanh(jnp.dot(x, win_ref[...]) + bin_ref[...] + r * (d * ch))
    h1 = (1.0 - ii) * nn
    h1_ref[...] = h1
    p = jax.nn.relu(jnp.dot(h1, o1_ref[...]) + ob1_ref[...])
    p = jax.nn.relu(jnp.dot(p, o2_ref[...]) + ob2_ref[...])
    p = jax.nn.relu(jnp.dot(p, o3_ref[...]) + ob3_ref[...])
    pred_ref[...] = x + p


def _edge_mlp_body(hrow_ref, hcol_ref, w1r_ref, w1c_ref, b1_ref,
                   w2_ref, b2_ref, out_ref):
    f32 = jnp.float32
    hr = hrow_ref[...].astype(jnp.bfloat16)
    hc = hcol_ref[...].astype(jnp.bfloat16)
    u = (jnp.dot(hr, w1r_ref[...], preferred_element_type=f32)
         + jnp.dot(hc, w1c_ref[...], preferred_element_type=f32)
         + b1_ref[...])
    m = jnp.tanh(u).astype(jnp.bfloat16)
    acc = None
    for i in range(3):
        t = jnp.tanh(jnp.dot(m[:, i * H:(i + 1) * H], w2_ref[i],
                             preferred_element_type=f32) + b2_ref[i:i + 1, :])
        acc = t if acc is None else acc + t
    out_ref[...] = acc


def _step1_body(x_ref, h_ref, *refs, n_agg):
    (whr_ref, whi_ref, whh_ref,
     wir_ref, bir_ref, wii_ref, bii_ref, win_ref, bin_ref,
     o1_ref, ob1_ref, o2_ref, ob2_ref, o3_ref, ob3_ref,
     pred_ref) = refs[n_agg:]
    agg = refs[0][...]
    for a in refs[1:n_agg]:
        agg = agg + a[...]
    x = x_ref[...]
    h = h_ref[...]
    r = _sigmoid(jnp.dot(x, wir_ref[...]) + bir_ref[...] + jnp.dot(agg, whr_ref[...]))
    ii = _sigmoid(jnp.dot(x, wii_ref[...]) + bii_ref[...] + jnp.dot(agg, whi_ref[...]))
    nn = jnp.tanh(jnp.dot(x, win_ref[...]) + bin_ref[...]
                  + r * jnp.dot(agg, whh_ref[...]))
    h2 = (1.0 - ii) * nn + ii * h
    p = jax.nn.relu(jnp.dot(h2, o1_ref[...]) + ob1_ref[...])
    p = jax.nn.relu(jnp.dot(p, o2_ref[...]) + ob2_ref[...])
    p = jax.nn.relu(jnp.dot(p, o3_ref[...]) + ob3_ref[...])
    pred_ref[...] = x + p


def _full(shape):
    return pl.BlockSpec(shape, lambda i: (0,) * len(shape))


def _rows(bn, w):
    return pl.BlockSpec((bn, w), lambda i: (i, 0))


def kernel(inputs, edge_index, burn_in_steps, params):
    _, _, N, _ = inputs.shape
    E = edge_index.shape[1]
    p = params

    e_pad = ((E + NW * CHUNK - 1) // (NW * CHUNK)) * (NW * CHUNK)
    n_chunks = e_pad // CHUNK
    n_pad = ((N + DW) + 63) // 64 * 64
    pad = e_pad - E

    row = edge_index[0]
    col = edge_index[1]
    row_p = jnp.concatenate([row, jnp.zeros((pad,), jnp.int32)]).reshape(n_chunks, CHUNK)
    col_p = jnp.concatenate([col, jnp.full((pad,), N, jnp.int32)]).reshape(n_chunks, CHUNK)

    ones_h = jnp.ones((CHUNK, H), jnp.float32)
    zeros_nh = jnp.zeros((n_pad, H), jnp.float32)

    deg2 = _make_hist_kernel(n_chunks, n_pad)(col_p, ones_h, zeros_nh)
    dega = deg2[0, :N, :DW]
    degb = deg2[1, :N, :DW]

    x0 = inputs[0, 0]

    BN = 2000
    grid = (N // BN,)
    wb = [
        p["msg_fc1_b"], p["msg_fc2_W"], p["msg_fc2_b"],
        p["hidden_r_W"], p["hidden_i_W"], p["hidden_h_W"],
        p["input_r_W"], p["input_r_b"].reshape(1, H),
        p["input_i_W"], p["input_i_b"].reshape(1, H),
        p["input_n_W"], p["input_n_b"].reshape(1, H),
        p["out_fc1_W"], p["out_fc1_b"].reshape(1, H),
        p["out_fc2_W"], p["out_fc2_b"].reshape(1, H),
        p["out_fc3_W"], p["out_fc3_b"].reshape(1, IN),
    ]
    wb_specs = [_full(w.shape) for w in wb]
    h1, pred0 = pl.pallas_call(
        _step0_body,
        grid=grid,
        in_specs=[_rows(BN, IN), _rows(BN, DW), _rows(BN, DW)] + wb_specs,
        out_specs=[_rows(BN, H), _rows(BN, IN)],
        out_shape=[
            jax.ShapeDtypeStruct((N, H), jnp.float32),
            jax.ShapeDtypeStruct((N, IN), jnp.float32),
        ],
    )(x0, dega, degb, *wb)

    bf16 = jnp.bfloat16
    w1 = p["msg_fc1_W"]
    w1r = jnp.concatenate([w1[i, :H, :] for i in range(3)], axis=1).astype(bf16)
    w1c = jnp.concatenate([w1[i, H:, :] for i in range(3)], axis=1).astype(bf16)
    b1f = p["msg_fc1_b"].reshape(1, 3 * H)
    w2b = p["msg_fc2_W"].astype(bf16)
    BE = 1024
    splits = [128, 256, 384, 512]
    assert sum(splits) == n_chunks
    aggs = []
    c0 = 0
    for si, n_sp in enumerate(splits):
        e_sp = n_sp * CHUNK
        hrow, hcol = _make_gather_kernel(n_sp, e_sp, H, (si + 1) % 2, c0)(
            h1, row_p, col_p)
        msgs = pl.pallas_call(
            _edge_mlp_body,
            grid=(e_sp // BE,),
            in_specs=[_rows(BE, H), _rows(BE, H), _full((H, 3 * H)),
                      _full((H, 3 * H)), _full((1, 3 * H)),
                      _full((3, H, H)), _full((3, H))],
            out_specs=[_rows(BE, H)],
            out_shape=[jax.ShapeDtypeStruct((e_sp, H), jnp.float32)],
        )(hrow, hcol, w1r, w1c, b1f, w2b, p["msg_fc2_b"])[0]
        agg2 = _make_scatter_kernel(n_sp, n_pad, c0)(msgs, col_p, zeros_nh)
        aggs += [agg2[0, :N, :], agg2[1, :N, :]]
        c0 += n_sp

    x1 = jnp.where(jnp.asarray(burn_in_steps) > 1, inputs[0, 1], pred0)
    wb1 = wb[3:]
    wb1_specs = [_full(w.shape) for w in wb1]
    pred1 = pl.pallas_call(
        functools.partial(_step1_body, n_agg=len(aggs)),
        grid=grid,
        in_specs=[_rows(BN, IN), _rows(BN, H)]
        + [_rows(BN, H)] * len(aggs) + wb1_specs,
        out_specs=[_rows(BN, IN)],
        out_shape=[jax.ShapeDtypeStruct((N, IN), jnp.float32)],
    )(x1, h1, *aggs, *wb1)[0]

    return jnp.stack([pred0, pred1], axis=0)[None]
